# baseline (speedup 1.0000x reference)
import jax
import jax.numpy as jnp
from jax import lax
from jax.experimental import pallas as pl
from jax.experimental.pallas import tpu as pltpu
from jax.experimental.pallas import tpu_sc as plsc

_MAX_REL = 32
_TBL = 2 * _MAX_REL + 1 + 1
_EMB = 16
_L = 16
_NW = 32
_WPAD = 80


def _body(off_hbm, msk_hbm, wt_hbm, b_hbm, out_hbm,
          off_v0, off_v1, m_v0, m_v1, wbt, trep, ob0, ob1,
          s_in0, s_in1, s_out0, s_out1):
    nj = off_hbm.shape[2]
    nrows = off_hbm.shape[0] * off_hbm.shape[1]
    rows_pw = nrows // _NW
    nk = off_hbm.shape[1]
    wid = lax.axis_index("s") * 2 + lax.axis_index("c")
    i0 = wid // (nk // rows_pw)
    k0 = (wid % (nk // rows_pw)) * rows_pw

    offs = [off_v0, off_v1]
    ms = [m_v0, m_v1]
    obs = [ob0, ob1]
    sin = [s_in0, s_in1]
    sout = [s_out0, s_out1]

    def start_in(k):
        p = k & 1
        h1 = pltpu.async_copy(off_hbm.at[i0, k0 + k, :], offs[p], sin[p])
        h2 = pltpu.async_copy(msk_hbm.at[i0, k0 + k, :], ms[p], sin[p])
        return (h1, h2)

    in_h = [start_in(0), start_in(1)]

    pltpu.sync_copy(wt_hbm, wbt.at[pl.ds(0, _EMB * _WPAD)])
    pltpu.sync_copy(b_hbm, wbt.at[pl.ds(_EMB * _WPAD, _EMB)])
    for c in range(_EMB):
        bc = plsc.load_gather(
            wbt, [jnp.full((_L,), _EMB * _WPAD + c, jnp.int32)])
        for blk in range(_WPAD // _L):
            s = pl.ds(c * _WPAD + blk * _L, _L)
            wbt[s] = wbt[s] + bc

    def rep_body(dd, carry):
        for c in range(_EMB):
            sp = plsc.load_gather(
                wbt, [jnp.full((_L,), c * _WPAD, jnp.int32) + dd])
            trep[pl.ds(c * (_TBL * _L) + dd * _L, _L)] = sp
        return carry

    lax.fori_loop(0, _TBL, rep_body, 0)

    lanes = lax.iota(jnp.int32, _L)
    out_h = [None, None]

    for k in range(rows_pw):
        p = k & 1
        in_h[p][0].wait()
        in_h[p][1].wait()
        if out_h[p] is not None:
            out_h[p].wait()
        off_v, m_v, ob = offs[p], ms[p], obs[p]

        @plsc.parallel_loop(0, nj // (2 * _L), unroll=1)
        def jb_body(jb):
            sl0 = pl.ds(jb * 2 * _L, _L)
            sl1 = pl.ds(jb * 2 * _L + _L, _L)
            off0, m0 = off_v[sl0], m_v[sl0]
            off1, m1 = off_v[sl1], m_v[sl1]
            t0 = jnp.minimum(jnp.maximum(off0 + _MAX_REL, 0), 2 * _MAX_REL)
            d0 = jnp.where(m0 == 0, 2 * _MAX_REL + 1, t0) * _L + lanes
            t1 = jnp.minimum(jnp.maximum(off1 + _MAX_REL, 0), 2 * _MAX_REL)
            d1 = jnp.where(m1 == 0, 2 * _MAX_REL + 1, t1) * _L + lanes
            g0 = [plsc.load_gather(trep, [d0 + c * (_TBL * _L)])
                  for c in range(_EMB)]
            g1 = [plsc.load_gather(trep, [d1 + c * (_TBL * _L)])
                  for c in range(_EMB)]
            for c in range(_EMB):
                ob[c, sl0] = g0[c]
                ob[c, sl1] = g1[c]

        out_h[p] = pltpu.async_copy(ob, out_hbm.at[i0, k0 + k], sout[p])
        if k + 2 < rows_pw:
            in_h[p] = start_in(k + 2)

    for h in out_h:
        if h is not None:
            h.wait()


def kernel(offset, mask, W, b):
    sb, sj, sk = offset.shape
    offt = jnp.transpose(offset, (0, 2, 1))
    mskt = jnp.transpose(mask, (0, 2, 1))
    Wt = jnp.pad(jnp.transpose(W), ((0, 0), (0, _WPAD - _TBL))).reshape(-1)
    mesh = plsc.VectorSubcoreMesh(core_axis_name="c", subcore_axis_name="s")
    f = pl.kernel(
        _body,
        out_type=jax.ShapeDtypeStruct((sb, sk, _EMB, sj), jnp.float32),
        mesh=mesh,
        scratch_types=[
            pltpu.VMEM((sj,), jnp.int32),
            pltpu.VMEM((sj,), jnp.int32),
            pltpu.VMEM((sj,), jnp.int32),
            pltpu.VMEM((sj,), jnp.int32),
            pltpu.VMEM((_EMB * _WPAD + _EMB,), jnp.float32),
            pltpu.VMEM((_EMB * _TBL * _L,), jnp.float32),
            pltpu.VMEM((_EMB, sj), jnp.float32),
            pltpu.VMEM((_EMB, sj), jnp.float32),
            pltpu.SemaphoreType.DMA,
            pltpu.SemaphoreType.DMA,
            pltpu.SemaphoreType.DMA,
            pltpu.SemaphoreType.DMA,
        ],
        compiler_params=pltpu.CompilerParams(
            needs_layout_passes=False, disable_bounds_checks=True),
    )
    out_t = f(offt, mskt, Wt, b)
    return jnp.transpose(out_t, (0, 3, 1, 2))


# speedup vs baseline: 1.0213x; 1.0213x over previous
import jax
import jax.numpy as jnp
from jax import lax
from jax.experimental import pallas as pl
from jax.experimental.pallas import tpu as pltpu
from jax.experimental.pallas import tpu_sc as plsc

_MAX_REL = 32
_TBL = 2 * _MAX_REL + 1 + 1
_EMB = 16
_L = 16
_NW = 32
_WPAD = 80


def _body(off_hbm, msk_hbm, wt_hbm, b_hbm, out_hbm,
          off_v0, off_v1, m_v0, m_v1, wbt, ob0, ob1,
          s_in0, s_in1, s_out0, s_out1):
    nj = off_hbm.shape[2]
    nrows = off_hbm.shape[0] * off_hbm.shape[1]
    rows_pw = nrows // _NW
    nk = off_hbm.shape[1]
    wid = lax.axis_index("s") * 2 + lax.axis_index("c")
    i0 = wid // (nk // rows_pw)
    k0 = (wid % (nk // rows_pw)) * rows_pw

    offs = [off_v0, off_v1]
    ms = [m_v0, m_v1]
    obs = [ob0, ob1]
    sin = [s_in0, s_in1]
    sout = [s_out0, s_out1]

    def start_in(k):
        p = k & 1
        h1 = pltpu.async_copy(off_hbm.at[i0, k0 + k, :], offs[p], sin[p])
        h2 = pltpu.async_copy(msk_hbm.at[i0, k0 + k, :], ms[p], sin[p])
        return (h1, h2)

    in_h = [start_in(0), start_in(1)]

    pltpu.sync_copy(wt_hbm, wbt.at[pl.ds(0, _EMB * _WPAD)])
    pltpu.sync_copy(b_hbm, wbt.at[pl.ds(_EMB * _WPAD, _EMB)])
    for c in range(_EMB):
        bc = plsc.load_gather(
            wbt, [jnp.full((_L,), _EMB * _WPAD + c, jnp.int32)])
        for blk in range(_WPAD // _L):
            s = pl.ds(c * _WPAD + blk * _L, _L)
            wbt[s] = wbt[s] + bc

    out_h = [None, None]

    for k in range(rows_pw):
        p = k & 1
        in_h[p][0].wait()
        in_h[p][1].wait()
        if out_h[p] is not None:
            out_h[p].wait()
        off_v, m_v, ob = offs[p], ms[p], obs[p]

        @plsc.parallel_loop(0, nj // (2 * _L), unroll=1)
        def jb_body(jb):
            sl0 = pl.ds(jb * 2 * _L, _L)
            sl1 = pl.ds(jb * 2 * _L + _L, _L)
            off0, m0 = off_v[sl0], m_v[sl0]
            off1, m1 = off_v[sl1], m_v[sl1]
            t0 = jnp.minimum(jnp.maximum(off0 + _MAX_REL, 0), 2 * _MAX_REL)
            d0 = jnp.where(m0 == 0, 2 * _MAX_REL + 1, t0)
            t1 = jnp.minimum(jnp.maximum(off1 + _MAX_REL, 0), 2 * _MAX_REL)
            d1 = jnp.where(m1 == 0, 2 * _MAX_REL + 1, t1)
            g0 = [plsc.load_gather(wbt, [d0 + c * _WPAD]) for c in range(_EMB)]
            g1 = [plsc.load_gather(wbt, [d1 + c * _WPAD]) for c in range(_EMB)]
            for c in range(_EMB):
                ob[c, sl0] = g0[c]
                ob[c, sl1] = g1[c]

        out_h[p] = pltpu.async_copy(ob, out_hbm.at[i0, k0 + k], sout[p])
        if k + 2 < rows_pw:
            in_h[p] = start_in(k + 2)

    for h in out_h:
        if h is not None:
            h.wait()


def kernel(offset, mask, W, b):
    sb, sj, sk = offset.shape
    offt = jnp.transpose(offset, (0, 2, 1))
    mskt = jnp.transpose(mask, (0, 2, 1))
    Wt = jnp.pad(jnp.transpose(W), ((0, 0), (0, _WPAD - _TBL))).reshape(-1)
    mesh = plsc.VectorSubcoreMesh(core_axis_name="c", subcore_axis_name="s")
    f = pl.kernel(
        _body,
        out_type=jax.ShapeDtypeStruct((sb, sk, _EMB, sj), jnp.float32),
        mesh=mesh,
        scratch_types=[
            pltpu.VMEM((sj,), jnp.int32),
            pltpu.VMEM((sj,), jnp.int32),
            pltpu.VMEM((sj,), jnp.int32),
            pltpu.VMEM((sj,), jnp.int32),
            pltpu.VMEM((_EMB * _WPAD + _EMB,), jnp.float32),
            pltpu.VMEM((_EMB, sj), jnp.float32),
            pltpu.VMEM((_EMB, sj), jnp.float32),
            pltpu.SemaphoreType.DMA,
            pltpu.SemaphoreType.DMA,
            pltpu.SemaphoreType.DMA,
            pltpu.SemaphoreType.DMA,
        ],
        compiler_params=pltpu.CompilerParams(
            needs_layout_passes=False, disable_bounds_checks=True),
    )
    out_t = f(offt, mskt, Wt, b)
    return jnp.transpose(out_t, (0, 3, 1, 2))
